# baseline (speedup 1.0000x reference)
import functools

import jax
import jax.numpy as jnp
from jax import lax
from jax.experimental import pallas as pl
from jax.experimental.pallas import tpu as pltpu
from jax.experimental.pallas import tpu_sc as plsc

NUM_BINS = 15
EPS = 1e-9
ROWS = 16384
COLS = 1000

_NC = 2
_NS = 16
_NW = _NC * _NS
_L = 16
_CR = 16


def _tc_body(x_ref, u_ref, cal_ref, o_ref):
    x = x_ref[...]
    m = jnp.max(x, axis=1, keepdims=True)
    e = jnp.exp(x - m)
    s = jnp.sum(e, axis=1, keepdims=True)
    p = jnp.clip(e / s, EPS, 1.0 - EPS)
    acc = jnp.full(x.shape, cal_ref[0], dtype=jnp.float32)
    for j in range(NUM_BINS - 1):
        acc = acc + jnp.where(p >= u_ref[j], cal_ref[j + 1], 0.0)
    o_ref[...] = acc


def _tc_calibrate(logits, u, cal, block_rows):
    grid = logits.shape[0] // block_rows
    return pl.pallas_call(
        _tc_body,
        grid=(grid,),
        in_specs=[
            pl.BlockSpec((block_rows, COLS), lambda i: (i, 0)),
            pl.BlockSpec(memory_space=pltpu.SMEM),
            pl.BlockSpec(memory_space=pltpu.SMEM),
        ],
        out_specs=pl.BlockSpec((block_rows, COLS), lambda i: (i, 0)),
        out_shape=jax.ShapeDtypeStruct(logits.shape, jnp.float32),
    )(logits, u, cal)


def _sc_calibrate(logits_flat, u16, cal16, n_rows, interpret=False):
    rpw = n_rows // _NW
    nk = rpw // _CR
    cw = _CR * COLS

    def body(x_hbm, u_hbm, cal_hbm, o_hbm, inb, xt, outb, uv, calv):
        wid = lax.axis_index("s") * _NC + lax.axis_index("c")
        pltpu.sync_copy(u_hbm, uv)
        pltpu.sync_copy(cal_hbm, calv)
        lane = lax.iota(jnp.int32, 16)
        fidx0 = lane * COLS
        uvec = uv[...]
        calvec = calv[...]
        t0 = calvec[0]

        def chunk(k, carry0):
            w0 = (wid * nk + k) * cw
            pltpu.sync_copy(x_hbm.at[pl.ds(w0, cw)], inb)

            def passT(ci, carry):
                m, fidx = carry
                for s in range(8):
                    v = plsc.load_gather(inb, [fidx])
                    xt[pl.ds((ci * 8 + s) * _L, _L)] = v
                    m = jnp.maximum(m, v)
                    fidx = fidx + 1
                return m, fidx
            m, _ = lax.fori_loop(0, COLS // 8, passT,
                                 (jnp.full((_L,), -jnp.inf, jnp.float32), fidx0))

            def passB(ci, s_acc):
                for s in range(8):
                    off = (ci * 8 + s) * _L
                    e = jnp.exp(xt[pl.ds(off, _L)] - m)
                    xt[pl.ds(off, _L)] = e
                    s_acc = s_acc + e
                return s_acc
            s_vec = lax.fori_loop(0, COLS // 8, passB,
                                  jnp.zeros((_L,), jnp.float32))

            elo = s_vec * jnp.float32(EPS)
            ehi = s_vec * jnp.float32(1.0 - EPS)
            vth = [uvec[j] * s_vec for j in range(NUM_BINS - 1)]
            dt = [calvec[j + 1] for j in range(NUM_BINS - 1)]

            def passC(ci, fidx):
                for s in range(4):
                    off = (ci * 4 + s) * _L
                    ec = jnp.minimum(jnp.maximum(xt[pl.ds(off, _L)], elo), ehi)
                    acc = jnp.full((_L,), t0, jnp.float32)
                    for j in range(NUM_BINS - 1):
                        acc = acc + jnp.where(ec >= vth[j], dt[j], 0.0)
                    plsc.store_scatter(outb, [fidx], acc)
                    fidx = fidx + 1
                return fidx
            lax.fori_loop(0, COLS // 4, passC, fidx0)

            pltpu.sync_copy(outb, o_hbm.at[pl.ds(w0, cw)])
            return carry0

        lax.fori_loop(0, nk, chunk, 0)

    return pl.kernel(
        body,
        out_type=jax.ShapeDtypeStruct((n_rows * COLS,), jnp.float32),
        mesh=plsc.VectorSubcoreMesh(core_axis_name="c", subcore_axis_name="s"),
        compiler_params=pltpu.CompilerParams(needs_layout_passes=False),
        scratch_types=[
            pltpu.VMEM((cw,), jnp.float32),
            pltpu.VMEM((cw,), jnp.float32),
            pltpu.VMEM((cw,), jnp.float32),
            pltpu.VMEM((16,), jnp.float32),
            pltpu.VMEM((16,), jnp.float32),
        ],
        interpret=interpret,
    )(logits_flat, u16, cal16)


@jax.jit
def kernel(logits, bin_boundaries, bin_reprs):
    u = jax.nn.sigmoid(bin_boundaries)
    t = jax.nn.sigmoid(bin_reprs)
    cal = jnp.concatenate([t[:1], jnp.diff(t)])
    u16 = jnp.pad(u, (0, 2))
    cal16 = jnp.pad(cal, (0, 1))
    out_flat = _sc_calibrate(logits.reshape(-1), u16, cal16, ROWS)
    return out_flat.reshape(logits.shape)


# speedup vs baseline: 1.2798x; 1.2798x over previous
import functools

import jax
import jax.numpy as jnp
from jax import lax
from jax.experimental import pallas as pl
from jax.experimental.pallas import tpu as pltpu
from jax.experimental.pallas import tpu_sc as plsc

NUM_BINS = 15
EPS = 1e-9
ROWS = 16384
COLS = 1000

_NC = 2
_NS = 16
_NW = _NC * _NS
_L = 16
_CR = 16


def _tc_body(x_ref, u_ref, cal_ref, o_ref):
    x = x_ref[...]
    m = jnp.max(x, axis=1, keepdims=True)
    e = jnp.exp(x - m)
    s = jnp.sum(e, axis=1, keepdims=True)
    p = jnp.clip(e / s, EPS, 1.0 - EPS)
    acc = jnp.full(x.shape, cal_ref[0], dtype=jnp.float32)
    for j in range(NUM_BINS - 1):
        acc = acc + jnp.where(p >= u_ref[j], cal_ref[j + 1], 0.0)
    o_ref[...] = acc


def _tc_calibrate(logits, u, cal, block_rows):
    grid = logits.shape[0] // block_rows
    return pl.pallas_call(
        _tc_body,
        grid=(grid,),
        in_specs=[
            pl.BlockSpec((block_rows, COLS), lambda i: (i, 0)),
            pl.BlockSpec(memory_space=pltpu.SMEM),
            pl.BlockSpec(memory_space=pltpu.SMEM),
        ],
        out_specs=pl.BlockSpec((block_rows, COLS), lambda i: (i, 0)),
        out_shape=jax.ShapeDtypeStruct(logits.shape, jnp.float32),
    )(logits, u, cal)


def _sc_calibrate(logits_flat, u16, cal16, n_rows, interpret=False):
    rpw = n_rows // _NW
    nk = rpw // _CR
    cw = _CR * COLS

    def body(x_hbm, u_hbm, cal_hbm, o_hbm, inb, xt, outb, uv, calv):
        wid = lax.axis_index("s") * _NC + lax.axis_index("c")
        pltpu.sync_copy(u_hbm, uv)
        pltpu.sync_copy(cal_hbm, calv)
        lane = lax.iota(jnp.int32, 16)
        fidx0 = lane * COLS
        uvec = uv[...]
        calvec = calv[...]
        t0 = calvec[0]

        vt = [jnp.full((_L,), calvec[j], jnp.float32) for j in range(NUM_BINS)]

        def chunk(k, carry0):
            w0 = (wid * nk + k) * cw
            pltpu.sync_copy(x_hbm.at[pl.ds(w0, cw)], inb)

            def passT(ci, carry):
                m, fidx = carry
                for s in range(8):
                    v = plsc.load_gather(inb, [fidx])
                    xt[pl.ds((ci * 8 + s) * _L, _L)] = v
                    m = jnp.maximum(m, v)
                    fidx = fidx + 1
                return m, fidx
            m, _ = lax.fori_loop(0, COLS // 8, passT,
                                 (jnp.full((_L,), -jnp.inf, jnp.float32), fidx0))

            def passB(ci, s_acc):
                for s in range(8):
                    off = (ci * 8 + s) * _L
                    e = jnp.exp(xt[pl.ds(off, _L)] - m)
                    xt[pl.ds(off, _L)] = e
                    s_acc = s_acc + e
                return s_acc
            s_vec = lax.fori_loop(0, COLS // 8, passB,
                                  jnp.zeros((_L,), jnp.float32))

            elo = s_vec * jnp.float32(EPS)
            ehi = s_vec * jnp.float32(1.0 - EPS)
            vth = [uvec[j] * s_vec for j in range(NUM_BINS - 1)]

            def passC(ci, fidx):
                for s in range(8):
                    off = (ci * 8 + s) * _L
                    ec = jnp.minimum(jnp.maximum(xt[pl.ds(off, _L)], elo), ehi)
                    acc = vt[0]
                    for j in range(NUM_BINS - 1):
                        acc = jnp.where(ec >= vth[j], vt[j + 1], acc)
                    plsc.store_scatter(outb, [fidx], acc)
                    fidx = fidx + 1
                return fidx
            lax.fori_loop(0, COLS // 8, passC, fidx0)

            pltpu.sync_copy(outb, o_hbm.at[pl.ds(w0, cw)])
            return carry0

        lax.fori_loop(0, nk, chunk, 0)

    return pl.kernel(
        body,
        out_type=jax.ShapeDtypeStruct((n_rows * COLS,), jnp.float32),
        mesh=plsc.VectorSubcoreMesh(core_axis_name="c", subcore_axis_name="s"),
        compiler_params=pltpu.CompilerParams(needs_layout_passes=False),
        scratch_types=[
            pltpu.VMEM((cw,), jnp.float32),
            pltpu.VMEM((cw,), jnp.float32),
            pltpu.VMEM((cw,), jnp.float32),
            pltpu.VMEM((16,), jnp.float32),
            pltpu.VMEM((16,), jnp.float32),
        ],
        interpret=interpret,
    )(logits_flat, u16, cal16)


@jax.jit
def kernel(logits, bin_boundaries, bin_reprs):
    u = jax.nn.sigmoid(bin_boundaries)
    t = jax.nn.sigmoid(bin_reprs)
    cal = jnp.concatenate([t[:1], jnp.diff(t)])
    u16 = jnp.pad(u, (0, 2))
    cal16 = jnp.pad(t, (0, 1))
    out_flat = _sc_calibrate(logits.reshape(-1), u16, cal16, ROWS)
    return out_flat.reshape(logits.shape)
